# baseline (speedup 1.0000x reference)
import functools

import jax
import jax.numpy as jnp
from jax import lax
from jax.experimental import pallas as pl
from jax.experimental.pallas import tpu as pltpu
from jax.experimental.pallas import tpu_sc as plsc

_B = 4096
_S = 26
_SP = 32
_DIM = 128
_NC = 2
_NS = 16
_NW = _NC * _NS
_CB = 4
_CI = _CB * _S
_NBUF = 4
_P = 4
_NBP = _B // _P
_RB = 8


@functools.cache
def _make_sc_kernel(nb):
    bpw = nb // _NW
    ipw = bpw * _S
    nchunk = bpw // _CB
    mesh = plsc.VectorSubcoreMesh(core_axis_name="c", subcore_axis_name="s")

    @functools.partial(
        pl.kernel,
        mesh=mesh,
        out_type=jax.ShapeDtypeStruct((nb * _SP, _DIM), jnp.float32),
        scratch_types=[
            pltpu.VMEM((ipw,), jnp.int32),
            pltpu.VMEM((_NBUF, _CB * _SP, _DIM), jnp.float32),
            pltpu.SemaphoreType.DMA((_NBUF,)),
            pltpu.SemaphoreType.DMA((_NBUF,)),
        ],
    )
    def gather_kernel(table_hbm, idx_hbm, out_hbm, idx_v, bufs, gsem, ssem):
        wid = lax.axis_index("s") * _NC + lax.axis_index("c")
        b0 = wid * bpw
        pltpu.sync_copy(idx_hbm.at[pl.ds(wid * ipw, ipw)], idx_v)

        def g_start(j, b):
            pltpu.async_copy(
                table_hbm.at[idx_v.at[pl.ds(j * _CI, _CI)]],
                bufs.at[b].at[pl.ds(0, _CI)],
                gsem.at[b],
            )

        def g_wait(b):
            pltpu.make_async_copy(
                table_hbm.at[pl.ds(0, _CI)],
                bufs.at[b].at[pl.ds(0, _CI)],
                gsem.at[b],
            ).wait()

        def s_start(j, b):
            for k in range(_CB):
                pltpu.async_copy(
                    bufs.at[b].at[pl.ds(k * _S, _SP)],
                    out_hbm.at[pl.ds((b0 + j * _CB + k) * _SP, _SP)],
                    ssem.at[b],
                )

        def s_wait(b):
            pltpu.make_async_copy(
                bufs.at[b], table_hbm.at[pl.ds(0, _CB * _SP)], ssem.at[b]
            ).wait()

        for b in range(_NBUF):
            g_start(b, b)

        def body(j0):
            for b in range(_NBUF):
                g_wait(b)
                s_start(j0 + b, b)

            for b in range(_NBUF):
                nxt = j0 + b + _NBUF

                @pl.when(nxt < nchunk)
                def _():
                    s_wait(b)
                    g_start(nxt, b)

        pl.loop(0, nchunk, step=_NBUF)(body)

        for b in range(_NBUF):
            s_wait(b)

    return gather_kernel


@jax.jit
def kernel(IX, weight):
    idx = IX.reshape(-1).astype(jnp.int32)
    sc = _make_sc_kernel(_NBP)
    chunks = [
        sc(weight, lax.dynamic_slice_in_dim(idx, p * _NBP * _S, _NBP * _S))
        for p in range(_P)
    ]
    one = (idx[0] * 0 + 1).astype(jnp.float32)
    out = jnp.zeros((_B, _S, _DIM), jnp.float32)
    for p in range(_P):
        c = chunks[p].reshape(_NBP, _SP, _DIM)[:, :_S, :] * one
        out = lax.dynamic_update_slice(out, c, (p * _NBP, 0, 0))
    return out


# speedup vs baseline: 1.0717x; 1.0717x over previous
import functools

import jax
import jax.numpy as jnp
from jax import lax
from jax.experimental import pallas as pl
from jax.experimental.pallas import tpu as pltpu
from jax.experimental.pallas import tpu_sc as plsc

_B = 4096
_S = 26
_SP = 32
_DIM = 128
_NC = 2
_NS = 16
_NW = _NC * _NS
_CB = 4
_CI = _CB * _S
_NBUF = 4
_P = 4
_NBP = _B // _P
_RB = 8


@functools.cache
def _make_sc_kernel(nb):
    bpw = nb // _NW
    ipw = bpw * _S
    nchunk = bpw // _CB
    mesh = plsc.VectorSubcoreMesh(core_axis_name="c", subcore_axis_name="s")

    @functools.partial(
        pl.kernel,
        mesh=mesh,
        out_type=jax.ShapeDtypeStruct((nb * _SP, _DIM), jnp.float32),
        scratch_types=[
            pltpu.VMEM((ipw,), jnp.int32),
            pltpu.VMEM((_NBUF, _CB * _SP, _DIM), jnp.float32),
            pltpu.SemaphoreType.DMA((_NBUF,)),
            pltpu.SemaphoreType.DMA((_NBUF,)),
        ],
    )
    def gather_kernel(table_hbm, idx_hbm, out_hbm, idx_v, bufs, gsem, ssem):
        wid = lax.axis_index("s") * _NC + lax.axis_index("c")
        b0 = wid * bpw
        pltpu.sync_copy(idx_hbm.at[pl.ds(wid * ipw, ipw)], idx_v)

        def g_start(j, b):
            pltpu.async_copy(
                table_hbm.at[idx_v.at[pl.ds(j * _CI, _CI)]],
                bufs.at[b].at[pl.ds(0, _CI)],
                gsem.at[b],
            )

        def g_wait(b):
            pltpu.make_async_copy(
                table_hbm.at[pl.ds(0, _CI)],
                bufs.at[b].at[pl.ds(0, _CI)],
                gsem.at[b],
            ).wait()

        def s_start(j, b):
            for k in range(_CB):
                pltpu.async_copy(
                    bufs.at[b].at[pl.ds(k * _S, _SP)],
                    out_hbm.at[pl.ds((b0 + j * _CB + k) * _SP, _SP)],
                    ssem.at[b],
                )

        def s_wait(b):
            pltpu.make_async_copy(
                bufs.at[b], table_hbm.at[pl.ds(0, _CB * _SP)], ssem.at[b]
            ).wait()

        for b in range(_NBUF):
            g_start(b, b)

        def body(j0):
            for b in range(_NBUF):
                g_wait(b)
                s_start(j0 + b, b)

            for b in range(_NBUF):
                nxt = j0 + b + _NBUF

                @pl.when(nxt < nchunk)
                def _():
                    s_wait(b)
                    g_start(nxt, b)

        pl.loop(0, nchunk, step=_NBUF)(body)

        for b in range(_NBUF):
            s_wait(b)

    return gather_kernel


@jax.jit
def kernel(IX, weight):
    idx = IX.reshape(-1).astype(jnp.int32)
    sc = _make_sc_kernel(_NBP)
    chunks = [
        sc(weight, lax.dynamic_slice_in_dim(idx, p * _NBP * _S, _NBP * _S))
        for p in range(_P)
    ]
    one = lax.optimization_barrier(jnp.float32(1.0))
    out = lax.optimization_barrier(jnp.zeros((_B, _S, _DIM), jnp.float32))
    for p in range(_P):
        c = chunks[p].reshape(_NBP, _SP, _DIM)[:, :_S, :] * one
        out = lax.dynamic_update_slice(out, c, (p * _NBP, 0, 0))
    return out


# speedup vs baseline: 1.3873x; 1.2945x over previous
import functools

import jax
import jax.numpy as jnp
from jax import lax
from jax.experimental import pallas as pl
from jax.experimental.pallas import tpu as pltpu
from jax.experimental.pallas import tpu_sc as plsc

_B = 4096
_S = 26
_DIM = 128
_NC = 2
_NS = 16
_NW = _NC * _NS
_BPW = _B // _NW
_IPW = _BPW * _S
_CB = 4
_CI = _CB * _S
_NCHUNK = _BPW // _CB
_NBUF = 4


@functools.cache
def _make_kernel():
    mesh = plsc.VectorSubcoreMesh(core_axis_name="c", subcore_axis_name="s")

    @functools.partial(
        pl.kernel,
        mesh=mesh,
        out_type=jax.ShapeDtypeStruct((_B, _S, _DIM), jnp.float32),
        scratch_types=[
            pltpu.VMEM((_IPW,), jnp.int32),
            pltpu.VMEM((_NBUF, _CI, _DIM), jnp.float32),
            pltpu.SemaphoreType.DMA((_NBUF,)),
            pltpu.SemaphoreType.DMA((_NBUF,)),
        ],
    )
    def gather_kernel(table_hbm, idx_hbm, out_hbm, idx_v, bufs, gsem, ssem):
        wid = lax.axis_index("s") * _NC + lax.axis_index("c")
        b0 = wid * _BPW
        pltpu.sync_copy(idx_hbm.at[pl.ds(wid * _IPW, _IPW)], idx_v)

        def g_start(j, b):
            pltpu.async_copy(
                table_hbm.at[idx_v.at[pl.ds(j * _CI, _CI)]], bufs.at[b], gsem.at[b]
            )

        def g_wait(b):
            pltpu.make_async_copy(
                table_hbm.at[pl.ds(0, _CI)], bufs.at[b], gsem.at[b]
            ).wait()

        def s_start(j, b):
            for k in range(_CB):
                pltpu.async_copy(
                    bufs.at[b].at[pl.ds(k * _S, _S)],
                    out_hbm.at[b0 + j * _CB + k],
                    ssem.at[b],
                )

        def s_wait(b):
            pltpu.make_async_copy(
                bufs.at[b], table_hbm.at[pl.ds(0, _CI)], ssem.at[b]
            ).wait()

        for b in range(_NBUF):
            g_start(b, b)

        def body(j0):
            for b in range(_NBUF):
                g_wait(b)
                s_start(j0 + b, b)

            for b in range(_NBUF):
                nxt = j0 + b + _NBUF

                @pl.when(nxt < _NCHUNK)
                def _():
                    s_wait(b)
                    g_start(nxt, b)

        pl.loop(0, _NCHUNK, step=_NBUF)(body)

        for b in range(_NBUF):
            s_wait(b)

    return gather_kernel


@jax.jit
def kernel(IX, weight):
    idx = IX.reshape(-1).astype(jnp.int32)
    return _make_kernel()(weight, idx)
